# Initial kernel scaffold; baseline (speedup 1.0000x reference)
import jax
import jax.numpy as jnp
from jax.experimental import pallas as pl


def kernel(z_local, global2local, heads, rels, tails, relation_emb):
    raise NotImplementedError("write your pallas kernel here")


# speedup vs baseline: 2.9409x; 2.9409x over previous
import functools

import jax
import jax.numpy as jnp
from jax import lax
from jax.experimental import pallas as pl
from jax.experimental.pallas import tpu as pltpu
from jax.experimental.pallas import tpu_sc as plsc

NUM_NODES = 10000
NUM_TRIPLES = 320000
NUM_RELATIONS = 1000
DIM = 128

L = 16
NW = 32
B_PER_W = NUM_TRIPLES // NW
CHUNK = 80
N_CHUNKS = B_PER_W // CHUNK
GROUPS = CHUNK // L


def _distmult_body(z_hbm, g2l_hbm, heads_hbm, rels_hbm, tails_hbm, rel_hbm,
                   out_hbm,
                   g2l_v, hg_v, rg_v, tg_v, hi_v, ti_v,
                   hrow_v, rrow_v, trow_v, out_v, sem):
    wid = lax.axis_index("s") * 2 + lax.axis_index("c")
    wbase = wid * B_PER_W

    pltpu.sync_copy(g2l_hbm, g2l_v)

    @pl.loop(0, N_CHUNKS)
    def chunk_body(ci):
        base = wbase + ci * CHUNK
        pltpu.sync_copy(heads_hbm.at[pl.ds(base, CHUNK)], hg_v)
        pltpu.sync_copy(rels_hbm.at[pl.ds(base, CHUNK)], rg_v)
        pltpu.sync_copy(tails_hbm.at[pl.ds(base, CHUNK)], tg_v)

        for k in range(GROUPS):
            sl = pl.ds(k * L, L)
            hi_v[sl] = plsc.load_gather(g2l_v, [hg_v[sl]])
            ti_v[sl] = plsc.load_gather(g2l_v, [tg_v[sl]])

        cp_h = pltpu.make_async_copy(z_hbm.at[hi_v], hrow_v, sem)
        cp_t = pltpu.make_async_copy(z_hbm.at[ti_v], trow_v, sem)
        cp_r = pltpu.make_async_copy(rel_hbm.at[rg_v], rrow_v, sem)
        cp_h.start()
        cp_t.start()
        cp_r.start()
        cp_h.wait()
        cp_t.wait()
        cp_r.wait()

        for g in range(GROUPS):
            rows = lax.iota(jnp.int32, L) + (g * L)

            def d_body(dd, acc):
                col = jnp.full((L,), dd, dtype=jnp.int32)
                h = plsc.load_gather(hrow_v, [rows, col])
                r = plsc.load_gather(rrow_v, [rows, col])
                t = plsc.load_gather(trow_v, [rows, col])
                return acc + h * r * t

            acc = lax.fori_loop(0, DIM, d_body, jnp.zeros((L,), jnp.float32))
            out_v[pl.ds(g * L, L)] = acc

        pltpu.sync_copy(out_v, out_hbm.at[pl.ds(base, CHUNK)])


@jax.jit
def _distmult(z_local, g2l, heads, rels, tails, rel_emb):
    mesh = plsc.VectorSubcoreMesh(core_axis_name="c", subcore_axis_name="s")
    kfn = pl.kernel(
        _distmult_body,
        mesh=mesh,
        compiler_params=pltpu.CompilerParams(needs_layout_passes=False),
        out_type=jax.ShapeDtypeStruct((NUM_TRIPLES,), jnp.float32),
        scratch_types=[
            pltpu.VMEM((NUM_NODES,), jnp.int32),
            pltpu.VMEM((CHUNK,), jnp.int32),
            pltpu.VMEM((CHUNK,), jnp.int32),
            pltpu.VMEM((CHUNK,), jnp.int32),
            pltpu.VMEM((CHUNK,), jnp.int32),
            pltpu.VMEM((CHUNK,), jnp.int32),
            pltpu.VMEM((CHUNK, DIM), jnp.float32),
            pltpu.VMEM((CHUNK, DIM), jnp.float32),
            pltpu.VMEM((CHUNK, DIM), jnp.float32),
            pltpu.VMEM((CHUNK,), jnp.float32),
            pltpu.SemaphoreType.DMA,
        ],
    )
    return kfn(z_local, g2l, heads, rels, tails, rel_emb)


def kernel(z_local, global2local, heads, rels, tails, relation_emb):
    return _distmult(
        z_local,
        global2local.astype(jnp.int32),
        heads.astype(jnp.int32),
        rels.astype(jnp.int32),
        tails.astype(jnp.int32),
        relation_emb,
    )


# speedup vs baseline: 3.9043x; 1.3276x over previous
import functools

import jax
import jax.numpy as jnp
from jax import lax
from jax.experimental import pallas as pl
from jax.experimental.pallas import tpu as pltpu
from jax.experimental.pallas import tpu_sc as plsc

NUM_NODES = 10000
NUM_TRIPLES = 320000
NUM_RELATIONS = 1000
DIM = 128

L = 16
NW = 32
B_PER_W = NUM_TRIPLES // NW
CHUNK = 80
N_CHUNKS = B_PER_W // CHUNK
GROUPS = CHUNK // L
N_PAIRS = (N_CHUNKS - 1) // 2


def _distmult_body(z_hbm, g2l_hbm, heads_hbm, rels_hbm, tails_hbm, rel_hbm,
                   out_hbm,
                   g2l_v, hds_v, rls_v, tls_v,
                   hi0, ti0, ri0, hi1, ti1, ri1,
                   hrow0, rrow0, trow0, hrow1, rrow1, trow1,
                   out_v, sem0, sem1):
    wid = lax.axis_index("s") * 2 + lax.axis_index("c")
    wbase = wid * B_PER_W

    pltpu.sync_copy(g2l_hbm, g2l_v)
    pltpu.sync_copy(heads_hbm.at[pl.ds(wbase, B_PER_W)], hds_v)
    pltpu.sync_copy(rels_hbm.at[pl.ds(wbase, B_PER_W)], rls_v)
    pltpu.sync_copy(tails_hbm.at[pl.ds(wbase, B_PER_W)], tls_v)

    def amap(ci, hi, ti, ri):
        for k in range(GROUPS):
            src = pl.ds(ci * CHUNK + k * L, L)
            dst = pl.ds(k * L, L)
            hi[dst] = plsc.load_gather(g2l_v, [hds_v[src]])
            ti[dst] = plsc.load_gather(g2l_v, [tls_v[src]])
            ri[dst] = rls_v[src]

    def copies(hi, ti, ri, hrow, rrow, trow, sem):
        return (pltpu.make_async_copy(z_hbm.at[hi], hrow, sem),
                pltpu.make_async_copy(rel_hbm.at[ri], rrow, sem),
                pltpu.make_async_copy(z_hbm.at[ti], trow, sem))

    def start(*bufs):
        for cp in copies(*bufs):
            cp.start()

    def drain(*bufs):
        for cp in copies(*bufs):
            cp.wait()

    def compute(ci, hrow, rrow, trow):
        for g in range(GROUPS):
            rows = lax.iota(jnp.int32, L) + (g * L)

            @pl.loop(0, DIM, init_carry=jnp.zeros((L,), jnp.float32),
                     unroll=16)
            def acc(dd, c):
                col = jnp.full((L,), dd, dtype=jnp.int32)
                h = plsc.load_gather(hrow, [rows, col])
                r = plsc.load_gather(rrow, [rows, col])
                t = plsc.load_gather(trow, [rows, col])
                return c + h * r * t

            out_v[pl.ds(ci * CHUNK + g * L, L)] = acc

    set0 = (hi0, ti0, ri0, hrow0, rrow0, trow0, sem0)
    set1 = (hi1, ti1, ri1, hrow1, rrow1, trow1, sem1)

    amap(0, hi0, ti0, ri0)
    start(*set0)

    @pl.loop(0, N_PAIRS)
    def pair(j):
        c0 = 2 * j
        amap(c0 + 1, hi1, ti1, ri1)
        start(*set1)
        drain(*set0)
        compute(c0, hrow0, rrow0, trow0)
        amap(c0 + 2, hi0, ti0, ri0)
        start(*set0)
        drain(*set1)
        compute(c0 + 1, hrow1, rrow1, trow1)

    drain(*set0)
    compute(N_CHUNKS - 1, hrow0, rrow0, trow0)

    pltpu.sync_copy(out_v, out_hbm.at[pl.ds(wbase, B_PER_W)])


@jax.jit
def _distmult(z_local, g2l, heads, rels, tails, rel_emb):
    mesh = plsc.VectorSubcoreMesh(core_axis_name="c", subcore_axis_name="s")
    idx_t = pltpu.VMEM((CHUNK,), jnp.int32)
    row_t = pltpu.VMEM((CHUNK, DIM), jnp.float32)
    kfn = pl.kernel(
        _distmult_body,
        mesh=mesh,
        compiler_params=pltpu.CompilerParams(needs_layout_passes=False),
        out_type=jax.ShapeDtypeStruct((NUM_TRIPLES,), jnp.float32),
        scratch_types=[
            pltpu.VMEM((NUM_NODES,), jnp.int32),
            pltpu.VMEM((B_PER_W,), jnp.int32),
            pltpu.VMEM((B_PER_W,), jnp.int32),
            pltpu.VMEM((B_PER_W,), jnp.int32),
            idx_t, idx_t, idx_t,
            idx_t, idx_t, idx_t,
            row_t, row_t, row_t,
            row_t, row_t, row_t,
            pltpu.VMEM((B_PER_W,), jnp.float32),
            pltpu.SemaphoreType.DMA,
            pltpu.SemaphoreType.DMA,
        ],
    )
    return kfn(z_local, g2l, heads, rels, tails, rel_emb)


def kernel(z_local, global2local, heads, rels, tails, relation_emb):
    return _distmult(
        z_local,
        global2local.astype(jnp.int32),
        heads.astype(jnp.int32),
        rels.astype(jnp.int32),
        tails.astype(jnp.int32),
        relation_emb,
    )


# speedup vs baseline: 28.9350x; 7.4111x over previous
import functools

import jax
import jax.numpy as jnp
from jax import lax
from jax.experimental import pallas as pl
from jax.experimental.pallas import tpu as pltpu
from jax.experimental.pallas import tpu_sc as plsc

NUM_NODES = 10000
NUM_TRIPLES = 320000
NUM_RELATIONS = 1000
DIM = 128

L = 16
NW = 32
B_PER_W = NUM_TRIPLES // NW
CHUNK = 80
N_CHUNKS = B_PER_W // CHUNK
GROUPS = CHUNK // L
N_PAIRS = (N_CHUNKS - 1) // 2


def _distmult_body(z_hbm, g2l_hbm, heads_hbm, rels_hbm, tails_hbm, rel_hbm,
                   out_hbm,
                   g2l_v, hds_v, rls_v, tls_v,
                   hi0, ti0, ri0, hi1, ti1, ri1,
                   hrow0, rrow0, trow0, hrow1, rrow1, trow1,
                   out_v, sem0, sem1):
    wid = lax.axis_index("s") * 2 + lax.axis_index("c")
    wbase = wid * B_PER_W

    pltpu.sync_copy(g2l_hbm, g2l_v)
    pltpu.sync_copy(heads_hbm.at[pl.ds(wbase, B_PER_W)], hds_v)
    pltpu.sync_copy(rels_hbm.at[pl.ds(wbase, B_PER_W)], rls_v)
    pltpu.sync_copy(tails_hbm.at[pl.ds(wbase, B_PER_W)], tls_v)

    def amap(ci, hi, ti, ri):
        for k in range(GROUPS):
            src = pl.ds(ci * CHUNK + k * L, L)
            dst = pl.ds(k * L, L)
            hi[dst] = plsc.load_gather(g2l_v, [hds_v[src]])
            ti[dst] = plsc.load_gather(g2l_v, [tls_v[src]])
            ri[dst] = rls_v[src]

    def copies(hi, ti, ri, hrow, rrow, trow, sem):
        return (pltpu.make_async_copy(z_hbm.at[hi], hrow, sem),
                pltpu.make_async_copy(rel_hbm.at[ri], rrow, sem),
                pltpu.make_async_copy(z_hbm.at[ti], trow, sem))

    def start(*bufs):
        for cp in copies(*bufs):
            cp.start()

    def drain(*bufs):
        for cp in copies(*bufs):
            cp.wait()

    def compute(ci, hrow, rrow, trow):
        lane = lax.iota(jnp.int32, L)
        for g in range(GROUPS):
            rows = lane + (g * L)

            @pl.loop(0, DIM,
                     init_carry=(jnp.zeros((L,), jnp.float32), lane),
                     unroll=16)
            def acc(dd, carry):
                c, col = carry
                h = plsc.load_gather(hrow, [rows, col])
                r = plsc.load_gather(rrow, [rows, col])
                t = plsc.load_gather(trow, [rows, col])
                return c + h * r * t, (col + 1) & (DIM - 1)

            out_v[pl.ds(ci * CHUNK + g * L, L)] = acc[0]

    set0 = (hi0, ti0, ri0, hrow0, rrow0, trow0, sem0)
    set1 = (hi1, ti1, ri1, hrow1, rrow1, trow1, sem1)

    amap(0, hi0, ti0, ri0)
    start(*set0)

    @pl.loop(0, N_PAIRS)
    def pair(j):
        c0 = 2 * j
        amap(c0 + 1, hi1, ti1, ri1)
        start(*set1)
        drain(*set0)
        compute(c0, hrow0, rrow0, trow0)
        amap(c0 + 2, hi0, ti0, ri0)
        start(*set0)
        drain(*set1)
        compute(c0 + 1, hrow1, rrow1, trow1)

    drain(*set0)
    compute(N_CHUNKS - 1, hrow0, rrow0, trow0)

    pltpu.sync_copy(out_v, out_hbm.at[pl.ds(wbase, B_PER_W)])


@jax.jit
def _distmult(z_local, g2l, heads, rels, tails, rel_emb):
    mesh = plsc.VectorSubcoreMesh(core_axis_name="c", subcore_axis_name="s")
    idx_t = pltpu.VMEM((CHUNK,), jnp.int32)
    row_t = pltpu.VMEM((CHUNK, DIM), jnp.float32)
    kfn = pl.kernel(
        _distmult_body,
        mesh=mesh,
        compiler_params=pltpu.CompilerParams(needs_layout_passes=False),
        out_type=jax.ShapeDtypeStruct((NUM_TRIPLES,), jnp.float32),
        scratch_types=[
            pltpu.VMEM((NUM_NODES,), jnp.int32),
            pltpu.VMEM((B_PER_W,), jnp.int32),
            pltpu.VMEM((B_PER_W,), jnp.int32),
            pltpu.VMEM((B_PER_W,), jnp.int32),
            idx_t, idx_t, idx_t,
            idx_t, idx_t, idx_t,
            row_t, row_t, row_t,
            row_t, row_t, row_t,
            pltpu.VMEM((B_PER_W,), jnp.float32),
            pltpu.SemaphoreType.DMA,
            pltpu.SemaphoreType.DMA,
        ],
    )
    return kfn(z_local, g2l, heads, rels, tails, rel_emb)


def kernel(z_local, global2local, heads, rels, tails, relation_emb):
    return _distmult(
        z_local,
        global2local.astype(jnp.int32),
        heads.astype(jnp.int32),
        rels.astype(jnp.int32),
        tails.astype(jnp.int32),
        relation_emb,
    )
